# baseline (speedup 1.0000x reference)
import functools

import jax
import jax.numpy as jnp
from jax import lax
from jax.experimental import pallas as pl
from jax.experimental.pallas import tpu as pltpu
from jax.experimental.pallas import tpu_sc as plsc

N_NODES = 10000
N_REL = 4
N_EDGES_PER_REL = 80000
F = 128
N_BASIS = 2

NC = 2
NS = 16
L = 16

CHUNK = 64
NBLK = 15
N_BLOCKS = 21
N_CHUNKS = NBLK * N_BLOCKS
E_PER_TILE = CHUNK * N_CHUNKS
E_REL_PAD = E_PER_TILE * NS // N_REL
NROT = 3
AHEAD = 2
N_PAD = 10240
ROWS_PER_TILE = N_PAD // NS


def _splat(i):
    return jnp.full((L,), i, dtype=jnp.int32)


def _sc_body(inp_hbm, src_hbm, dst_hbm, w_hbm, coeff_hbm, zeros_hbm, z_hbm,
             src_buf, dst_buf, w_buf, coeff_v, rows_v, acc,
             gsem, ssem, isem):
    cid = lax.axis_index("c")
    sid = lax.axis_index("s")

    pltpu.sync_copy(coeff_hbm, coeff_v)
    row0 = sid * ROWS_PER_TILE
    pltpu.sync_copy(zeros_hbm.at[pl.ds(row0, ROWS_PER_TILE)],
                    acc.at[pl.ds(row0, ROWS_PER_TILE)])

    rel = sid // (NS // N_REL)
    cvec = plsc.load_gather(coeff_v, [_splat(rel * N_BASIS + cid)])
    plsc.subcore_barrier()

    def _issue_idx_loads(b, pb):
        pltpu.async_copy(src_hbm.at[sid, b], src_buf.at[pb], isem.at[0])
        pltpu.async_copy(dst_hbm.at[sid, b], dst_buf.at[pb], isem.at[1])
        pltpu.async_copy(w_hbm.at[sid, b], w_buf.at[pl.ds(pb * NBLK, NBLK)],
                         isem.at[2])

    _issue_idx_loads(0, 0)
    pltpu.make_async_copy(src_hbm.at[sid, 0], src_buf.at[0],
                          isem.at[0]).wait()
    for p0 in range(AHEAD):
        pltpu.async_copy(inp_hbm.at[src_buf.at[0].at[p0]], rows_v.at[p0],
                         gsem.at[p0])

    LASTG = NBLK // NROT - 1

    def _block(b, carry):
        pb = lax.rem(b, 2)
        pltpu.make_async_copy(dst_hbm.at[sid, b], dst_buf.at[pb],
                              isem.at[1]).wait()
        pltpu.make_async_copy(w_hbm.at[sid, b],
                              w_buf.at[pl.ds(pb * NBLK, NBLK)],
                              isem.at[2]).wait()

        @pl.when(b + 1 < N_BLOCKS)
        def _():
            _issue_idx_loads(b + 1, 1 - pb)

        sb = src_buf.at[pb]
        db = dst_buf.at[pb]
        sb_next = src_buf.at[1 - pb]

        def _gather(c, p):
            return pltpu.async_copy(inp_hbm.at[sb.at[c]], rows_v.at[p],
                                    gsem.at[p])

        def _group(g, carry2):
            for j in range(NROT):
                c = NROT * g + j
                p = j
                q = (j + AHEAD) % NROT
                pltpu.make_async_copy(inp_hbm.at[sb.at[c]], rows_v.at[p],
                                      gsem.at[p]).wait()
                wrow = pb * NBLK + c

                @plsc.parallel_loop(0, CHUNK, unroll=4)
                def _scale(e, p=p, wrow=wrow):
                    wv = plsc.load_gather(
                        w_buf, [_splat(wrow), _splat(e)]) * cvec
                    for jj in range(F // L):
                        sl = pl.ds(jj * L, L)
                        rows_v[p, e, sl] = rows_v[p, e, sl] * wv

                pltpu.async_copy(rows_v.at[p], acc.at[db.at[c]], ssem.at[p],
                                 add=True)
                if j == 0:
                    @pl.when((g > 0) | (b > 0))
                    def _():
                        pltpu.make_async_copy(
                            rows_v.at[q], acc.at[db.at[c - 1]],
                            ssem.at[q]).wait()
                    _gather(c + AHEAD, q)
                else:
                    @pl.when(g < LASTG)
                    def _():
                        pltpu.make_async_copy(
                            rows_v.at[q], acc.at[db.at[c - 1]],
                            ssem.at[q]).wait()
                        _gather(c + AHEAD, q)

                    @pl.when(g == LASTG)
                    def _():
                        pltpu.make_async_copy(
                            rows_v.at[q], acc.at[db.at[c - 1]],
                            ssem.at[q]).wait()

                        @pl.when(b + 1 < N_BLOCKS)
                        def _():
                            if j == 1:
                                pltpu.make_async_copy(
                                    src_hbm.at[sid, b + 1],
                                    src_buf.at[1 - pb], isem.at[0]).wait()
                            pltpu.async_copy(
                                inp_hbm.at[sb_next.at[j - 1]],
                                rows_v.at[q], gsem.at[q])
            return carry2

        lax.fori_loop(0, NBLK // NROT, _group, 0)
        return carry

    lax.fori_loop(0, N_BLOCKS, _block, 0)
    pltpu.make_async_copy(rows_v.at[NROT - 1],
                          acc.at[dst_buf.at[0].at[NBLK - 1]],
                          ssem.at[(NBLK - 1) % NROT]).wait()
    plsc.subcore_barrier()
    pltpu.sync_copy(acc.at[pl.ds(row0, ROWS_PER_TILE)],
                    z_hbm.at[cid, pl.ds(row0, ROWS_PER_TILE)])


@functools.partial(
    pl.kernel,
    out_type=jax.ShapeDtypeStruct((N_BASIS, N_PAD, F), jnp.float32),
    mesh=plsc.VectorSubcoreMesh(core_axis_name="c", subcore_axis_name="s"),
    scratch_types=[
        pltpu.VMEM((2, NBLK, CHUNK), jnp.int32),
        pltpu.VMEM((2, NBLK, CHUNK), jnp.int32),
        pltpu.VMEM((2 * NBLK, CHUNK), jnp.float32),
        pltpu.VMEM((L,), jnp.float32),
        pltpu.VMEM((NROT, CHUNK, F), jnp.float32),
        pltpu.VMEM_SHARED((N_PAD, F), jnp.float32),
        pltpu.SemaphoreType.DMA((NROT,)),
        pltpu.SemaphoreType.DMA((NROT,)),
        pltpu.SemaphoreType.DMA((3,)),
    ],
    compiler_params=pltpu.CompilerParams(needs_layout_passes=False),
)
def _sc_scatter(inp, src, dst, w, coeff, zeros, z_out,
                src_buf, dst_buf, w_buf, coeff_v, rows_v, acc,
                gsem, ssem, isem):
    _sc_body(inp, src, dst, w, coeff, zeros, z_out,
             src_buf, dst_buf, w_buf, coeff_v, rows_v, acc,
             gsem, ssem, isem)


ROW_BLK = 400


def _mm_body(z_ref, bw_ref, o_ref):
    o_ref[...] = (
        jnp.dot(z_ref[0], bw_ref[0], preferred_element_type=jnp.float32)
        + jnp.dot(z_ref[1], bw_ref[1], preferred_element_type=jnp.float32)
    )


def _basis_matmul(z, basis_weights):
    return pl.pallas_call(
        _mm_body,
        grid=(N_NODES // ROW_BLK,),
        in_specs=[
            pl.BlockSpec((N_BASIS, ROW_BLK, F), lambda i: (0, i, 0)),
            pl.BlockSpec((N_BASIS, F, F), lambda i: (0, 0, 0)),
        ],
        out_specs=pl.BlockSpec((ROW_BLK, F), lambda i: (i, 0)),
        out_shape=jax.ShapeDtypeStruct((N_NODES, F), jnp.float32),
    )(z, basis_weights)


def kernel(inp, edge_index, edge_weight, basis_coeff, basis_weights):
    pad = E_REL_PAD - N_EDGES_PER_REL
    dst = jnp.pad(edge_index[:, 0, :].astype(jnp.int32), ((0, 0), (0, pad)))
    src = jnp.pad(edge_index[:, 1, :].astype(jnp.int32), ((0, 0), (0, pad)))
    w = jnp.pad(edge_weight.astype(jnp.float32), ((0, 0), (0, pad)))
    dst = dst.reshape(NS, N_BLOCKS, NBLK, CHUNK)
    src = src.reshape(NS, N_BLOCKS, NBLK, CHUNK)
    w = w.reshape(NS, N_BLOCKS, NBLK, CHUNK)
    coeff = jnp.pad(basis_coeff.reshape(-1), (0, L - N_REL * N_BASIS))
    zeros = jnp.zeros((N_PAD, F), jnp.float32)
    z = _sc_scatter(inp, src, dst, w, coeff, zeros)
    return _basis_matmul(z, basis_weights)


# speedup vs baseline: 1.0325x; 1.0325x over previous
import functools

import jax
import jax.numpy as jnp
from jax import lax
from jax.experimental import pallas as pl
from jax.experimental.pallas import tpu as pltpu
from jax.experimental.pallas import tpu_sc as plsc

N_NODES = 10000
N_REL = 4
N_EDGES_PER_REL = 80000
F = 128
N_BASIS = 2

NC = 2
NS = 16
L = 16

CHUNK = 48
NBLK = 20
N_BLOCKS = 21
N_CHUNKS = NBLK * N_BLOCKS
E_PER_TILE = CHUNK * N_CHUNKS
E_REL_PAD = E_PER_TILE * NS // N_REL
NROT = 4
AHEAD = 3
N_PAD = 10240
ROWS_PER_TILE = N_PAD // NS


def _splat(i):
    return jnp.full((L,), i, dtype=jnp.int32)


def _sc_body(inp_hbm, src_hbm, dst_hbm, w_hbm, coeff_hbm, zeros_hbm, z_hbm,
             src_buf, dst_buf, w_buf, coeff_v, rows_v, acc,
             gsem, ssem, isem):
    cid = lax.axis_index("c")
    sid = lax.axis_index("s")

    pltpu.sync_copy(coeff_hbm, coeff_v)
    row0 = sid * ROWS_PER_TILE
    pltpu.sync_copy(zeros_hbm.at[pl.ds(row0, ROWS_PER_TILE)],
                    acc.at[pl.ds(row0, ROWS_PER_TILE)])

    rel = sid // (NS // N_REL)
    cvec = plsc.load_gather(coeff_v, [_splat(rel * N_BASIS + cid)])
    plsc.subcore_barrier()

    def _issue_idx_loads(b, pb):
        pltpu.async_copy(src_hbm.at[sid, b], src_buf.at[pb], isem.at[0])
        pltpu.async_copy(dst_hbm.at[sid, b], dst_buf.at[pb], isem.at[1])
        pltpu.async_copy(w_hbm.at[sid, b], w_buf.at[pl.ds(pb * NBLK, NBLK)],
                         isem.at[2])

    _issue_idx_loads(0, 0)
    pltpu.make_async_copy(src_hbm.at[sid, 0], src_buf.at[0],
                          isem.at[0]).wait()
    for p0 in range(AHEAD):
        pltpu.async_copy(inp_hbm.at[src_buf.at[0].at[p0]], rows_v.at[p0],
                         gsem.at[p0])

    LASTG = NBLK // NROT - 1

    def _block(b, carry):
        pb = lax.rem(b, 2)
        pltpu.make_async_copy(dst_hbm.at[sid, b], dst_buf.at[pb],
                              isem.at[1]).wait()
        pltpu.make_async_copy(w_hbm.at[sid, b],
                              w_buf.at[pl.ds(pb * NBLK, NBLK)],
                              isem.at[2]).wait()

        @pl.when(b + 1 < N_BLOCKS)
        def _():
            _issue_idx_loads(b + 1, 1 - pb)

        sb = src_buf.at[pb]
        db = dst_buf.at[pb]
        sb_next = src_buf.at[1 - pb]

        def _gather(c, p):
            return pltpu.async_copy(inp_hbm.at[sb.at[c]], rows_v.at[p],
                                    gsem.at[p])

        def _group(g, carry2):
            for j in range(NROT):
                c = NROT * g + j
                p = j
                q = (j + AHEAD) % NROT
                pltpu.make_async_copy(inp_hbm.at[sb.at[c]], rows_v.at[p],
                                      gsem.at[p]).wait()
                wrow = pb * NBLK + c

                @plsc.parallel_loop(0, CHUNK, unroll=4)
                def _scale(e, p=p, wrow=wrow):
                    wv = plsc.load_gather(
                        w_buf, [_splat(wrow), _splat(e)]) * cvec
                    for jj in range(F // L):
                        sl = pl.ds(jj * L, L)
                        rows_v[p, e, sl] = rows_v[p, e, sl] * wv

                pltpu.async_copy(rows_v.at[p], acc.at[db.at[c]], ssem.at[p],
                                 add=True)
                if j == 0:
                    @pl.when((g > 0) | (b > 0))
                    def _():
                        pltpu.make_async_copy(
                            rows_v.at[q], acc.at[db.at[c - 1]],
                            ssem.at[q]).wait()
                    _gather(c + AHEAD, q)
                else:
                    @pl.when(g < LASTG)
                    def _():
                        pltpu.make_async_copy(
                            rows_v.at[q], acc.at[db.at[c - 1]],
                            ssem.at[q]).wait()
                        _gather(c + AHEAD, q)

                    @pl.when(g == LASTG)
                    def _():
                        pltpu.make_async_copy(
                            rows_v.at[q], acc.at[db.at[c - 1]],
                            ssem.at[q]).wait()

                        @pl.when(b + 1 < N_BLOCKS)
                        def _():
                            if j == 1:
                                pltpu.make_async_copy(
                                    src_hbm.at[sid, b + 1],
                                    src_buf.at[1 - pb], isem.at[0]).wait()
                            pltpu.async_copy(
                                inp_hbm.at[sb_next.at[j - 1]],
                                rows_v.at[q], gsem.at[q])
            return carry2

        lax.fori_loop(0, NBLK // NROT, _group, 0)
        return carry

    lax.fori_loop(0, N_BLOCKS, _block, 0)
    pltpu.make_async_copy(rows_v.at[NROT - 1],
                          acc.at[dst_buf.at[0].at[NBLK - 1]],
                          ssem.at[(NBLK - 1) % NROT]).wait()
    plsc.subcore_barrier()
    pltpu.sync_copy(acc.at[pl.ds(row0, ROWS_PER_TILE)],
                    z_hbm.at[cid, pl.ds(row0, ROWS_PER_TILE)])


@functools.partial(
    pl.kernel,
    out_type=jax.ShapeDtypeStruct((N_BASIS, N_PAD, F), jnp.float32),
    mesh=plsc.VectorSubcoreMesh(core_axis_name="c", subcore_axis_name="s"),
    scratch_types=[
        pltpu.VMEM((2, NBLK, CHUNK), jnp.int32),
        pltpu.VMEM((2, NBLK, CHUNK), jnp.int32),
        pltpu.VMEM((2 * NBLK, CHUNK), jnp.float32),
        pltpu.VMEM((L,), jnp.float32),
        pltpu.VMEM((NROT, CHUNK, F), jnp.float32),
        pltpu.VMEM_SHARED((N_PAD, F), jnp.float32),
        pltpu.SemaphoreType.DMA((NROT,)),
        pltpu.SemaphoreType.DMA((NROT,)),
        pltpu.SemaphoreType.DMA((3,)),
    ],
    compiler_params=pltpu.CompilerParams(needs_layout_passes=False),
)
def _sc_scatter(inp, src, dst, w, coeff, zeros, z_out,
                src_buf, dst_buf, w_buf, coeff_v, rows_v, acc,
                gsem, ssem, isem):
    _sc_body(inp, src, dst, w, coeff, zeros, z_out,
             src_buf, dst_buf, w_buf, coeff_v, rows_v, acc,
             gsem, ssem, isem)


ROW_BLK = 400


def _mm_body(z_ref, bw_ref, o_ref):
    o_ref[...] = (
        jnp.dot(z_ref[0], bw_ref[0], preferred_element_type=jnp.float32)
        + jnp.dot(z_ref[1], bw_ref[1], preferred_element_type=jnp.float32)
    )


def _basis_matmul(z, basis_weights):
    return pl.pallas_call(
        _mm_body,
        grid=(N_NODES // ROW_BLK,),
        in_specs=[
            pl.BlockSpec((N_BASIS, ROW_BLK, F), lambda i: (0, i, 0)),
            pl.BlockSpec((N_BASIS, F, F), lambda i: (0, 0, 0)),
        ],
        out_specs=pl.BlockSpec((ROW_BLK, F), lambda i: (i, 0)),
        out_shape=jax.ShapeDtypeStruct((N_NODES, F), jnp.float32),
    )(z, basis_weights)


def kernel(inp, edge_index, edge_weight, basis_coeff, basis_weights):
    pad = E_REL_PAD - N_EDGES_PER_REL
    dst = jnp.pad(edge_index[:, 0, :].astype(jnp.int32), ((0, 0), (0, pad)))
    src = jnp.pad(edge_index[:, 1, :].astype(jnp.int32), ((0, 0), (0, pad)))
    w = jnp.pad(edge_weight.astype(jnp.float32), ((0, 0), (0, pad)))
    dst = dst.reshape(NS, N_BLOCKS, NBLK, CHUNK)
    src = src.reshape(NS, N_BLOCKS, NBLK, CHUNK)
    w = w.reshape(NS, N_BLOCKS, NBLK, CHUNK)
    coeff = jnp.pad(basis_coeff.reshape(-1), (0, L - N_REL * N_BASIS))
    zeros = jnp.zeros((N_PAD, F), jnp.float32)
    z = _sc_scatter(inp, src, dst, w, coeff, zeros)
    return _basis_matmul(z, basis_weights)
